# baseline (speedup 1.0000x reference)
import functools

import jax
import jax.numpy as jnp
from jax import lax
from jax.experimental import pallas as pl
from jax.experimental.pallas import tpu as pltpu
from jax.experimental.pallas import tpu_sc as plsc

_F = 26
_B = 16384
_EMBED = 32
_BC = 128


def _lookup(idx_t, table4):
    info = plsc.get_sparse_core_info()
    nw = info.num_cores * info.num_subcores
    units = _F * (_B // _BC)
    upw = units // nw

    mesh = plsc.VectorSubcoreMesh(core_axis_name="c", subcore_axis_name="s")

    @functools.partial(
        pl.kernel,
        mesh=mesh,
        out_type=jax.ShapeDtypeStruct((_F, _EMBED, _B), jnp.float32),
        scratch_types=[
            pltpu.VMEM((_BC,), jnp.int32),
            pltpu.VMEM((_BC,), jnp.int32),
            pltpu.VMEM((_BC,), jnp.int32),
            pltpu.VMEM((_BC, 128), jnp.float32),
            pltpu.VMEM((_EMBED, _BC), jnp.float32),
            pltpu.SemaphoreType.DMA,
        ],
        compiler_params=pltpu.CompilerParams(
            use_tc_tiling_on_sc=True, needs_layout_passes=False
        ),
    )
    def k(idx_hbm, tab_hbm, out_hbm, idxb, gidx, laneb, rows4, outb, gsem):
        wid = lax.axis_index("s") * info.num_cores + lax.axis_index("c")

        def unit(c, carry):
            u = c * nw + wid
            f = lax.rem(u, _F)
            b0 = lax.div(u, _F) * _BC
            pltpu.sync_copy(idx_hbm.at[f, pl.ds(b0, _BC)], idxb)
            for j in range(8):
                v = idxb[pl.ds(j * 16, 16)]
                gidx[pl.ds(j * 16, 16)] = lax.shift_right_logical(v, 2)
                laneb[pl.ds(j * 16, 16)] = (v & 3) * _EMBED
            pltpu.async_copy(tab_hbm.at[gidx], rows4, gsem).wait()
            for j in range(8):
                slot = jnp.arange(16, dtype=jnp.int32) + (j * 16)
                lb = laneb[pl.ds(j * 16, 16)]
                for e in range(_EMBED):
                    outb.at[e][pl.ds(j * 16, 16)] = plsc.load_gather(
                        rows4, [slot, lb + e]
                    )
            pltpu.sync_copy(outb, out_hbm.at[f, :, pl.ds(b0, _BC)])
            return carry

        lax.fori_loop(0, upw, unit, 0)

    return k(idx_t, table4)


def kernel(tensor, table):
    idx_t = tensor.T.astype(jnp.int32)
    table4 = table.reshape(250000, 128)
    out_t = _lookup(idx_t, table4)
    return out_t.transpose(2, 0, 1)


# speedup vs baseline: 1.2374x; 1.2374x over previous
import functools

import jax
import jax.numpy as jnp
from jax import lax
from jax.experimental import pallas as pl
from jax.experimental.pallas import tpu as pltpu
from jax.experimental.pallas import tpu_sc as plsc

_F = 26
_B = 16384
_EMBED = 32
_BC = 128
_W = 512
_SUB = _W // _BC
_UNITS = _F * _SUB
_NIDX = _F * _W


def _lookup(idx_t, table4):
    info = plsc.get_sparse_core_info()
    nw = info.num_cores * info.num_subcores
    assert nw * _W == _B

    mesh = plsc.VectorSubcoreMesh(core_axis_name="c", subcore_axis_name="s")

    @functools.partial(
        pl.kernel,
        mesh=mesh,
        out_type=jax.ShapeDtypeStruct((_F, _EMBED, _B), jnp.float32),
        scratch_types=[
            pltpu.VMEM((_NIDX,), jnp.int32),
            pltpu.VMEM((_NIDX,), jnp.int32),
            pltpu.VMEM((_NIDX,), jnp.int32),
            pltpu.VMEM((4 * _BC, 128), jnp.float32),
            pltpu.VMEM((2 * _EMBED, _BC), jnp.float32),
            pltpu.SemaphoreType.DMA,
            pltpu.SemaphoreType.DMA,
            pltpu.SemaphoreType.DMA,
        ],
        compiler_params=pltpu.CompilerParams(
            use_tc_tiling_on_sc=True, needs_layout_passes=False
        ),
    )
    def k(idx_hbm, tab_hbm, out_hbm, idxb, gidx, laneb, rows4, outb,
          isem, gsem, wsem):
        wid = lax.axis_index("s") * info.num_cores + lax.axis_index("c")
        b0w = wid * _W

        icps = [
            pltpu.async_copy(
                idx_hbm.at[f, pl.ds(b0w, _W)],
                idxb.at[pl.ds(f * _W, _W)],
                isem,
            )
            for f in range(_F)
        ]
        for cp in icps:
            cp.wait()

        def prep(i, carry):
            v = idxb[pl.ds(i * 16, 16)]
            gidx[pl.ds(i * 16, 16)] = lax.shift_right_logical(v, 2)
            laneb[pl.ds(i * 16, 16)] = (v & 3) * _EMBED
            return carry

        lax.fori_loop(0, _NIDX // 16, prep, 0)

        def unit_off(c):
            f = lax.rem(c, _F)
            sub = lax.div(c, _F)
            return f, sub, f * _W + sub * _BC

        def fire_gather(c, slot):
            _, _, off = unit_off(c)
            return pltpu.async_copy(
                tab_hbm.at[gidx.at[pl.ds(off, _BC)]],
                rows4.at[pl.ds(slot * _BC, _BC), :],
                gsem,
            )

        fire_gather(0, 0)
        fire_gather(1, 1)

        def body(cc, carry):
            for kk in range(4):
                c = cc * 4 + kk
                f, sub, off = unit_off(c)

                @pl.when(c + 2 < _UNITS)
                def _():
                    fire_gather(c + 2, (kk + 2) % 4)

                pltpu.make_async_copy(
                    tab_hbm.at[gidx.at[pl.ds(0, _BC)]],
                    rows4.at[pl.ds(0, _BC), :],
                    gsem,
                ).wait()

                @pl.when(c >= 2)
                def _():
                    pltpu.make_async_copy(
                        outb.at[pl.ds(0, _EMBED), :],
                        out_hbm.at[0, :, pl.ds(0, _BC)],
                        wsem,
                    ).wait()

                orow = (kk % 2) * _EMBED
                for j in range(_BC // 16):
                    rowvec = jnp.arange(16, dtype=jnp.int32) + (
                        kk * _BC + j * 16
                    )
                    lb = laneb[pl.ds(off + j * 16, 16)]
                    for e in range(_EMBED):
                        outb.at[orow + e][pl.ds(j * 16, 16)] = (
                            plsc.load_gather(rows4, [rowvec, lb + e])
                        )
                pltpu.async_copy(
                    outb.at[pl.ds(orow, _EMBED), :],
                    out_hbm.at[f, :, pl.ds(b0w + sub * _BC, _BC)],
                    wsem,
                )
            return carry

        lax.fori_loop(0, _UNITS // 4, body, 0)

        for _ in range(2):
            pltpu.make_async_copy(
                outb.at[pl.ds(0, _EMBED), :],
                out_hbm.at[0, :, pl.ds(0, _BC)],
                wsem,
            ).wait()

    return k(idx_t, table4)


def kernel(tensor, table):
    idx_t = tensor.T.astype(jnp.int32)
    table4 = table.reshape(250000, 128)
    out_t = _lookup(idx_t, table4)
    return out_t.transpose(2, 0, 1)
